# Initial kernel scaffold; baseline (speedup 1.0000x reference)
import jax
import jax.numpy as jnp
from jax.experimental import pallas as pl


def kernel(token_index, emb_table, fc_w, fc_b):
    raise NotImplementedError("write your pallas kernel here")


# speedup vs baseline: 10.6732x; 10.6732x over previous
import functools

import jax
import jax.numpy as jnp
from jax import lax
from jax.experimental import pallas as pl
from jax.experimental.pallas import tpu as pltpu
from jax.experimental.pallas import tpu_sc as plsc

_B = 16384
_H = 200
_D = 32
_NC = 2
_NS = 16
_NW = _NC * _NS
_BPW = _B // _NW
_CROWS = 16
_NCHUNK = _BPW // _CROWS
_TPC = _CROWS * _H
_IMINOR = 128
_IROWS = _TPC // _IMINOR
_IDX_ROWS_TOTAL = _B * _H // _IMINOR
_IDX_PER_W = _IDX_ROWS_TOTAL // _NW
_STAGE_IROWS = 200
_CHUNKS_PER_STAGE = _STAGE_IROWS // _IROWS
_NSTAGE = _IDX_PER_W // _STAGE_IROWS


def _make_sc_kernel():
  mesh = plsc.VectorSubcoreMesh(core_axis_name="c", subcore_axis_name="s")

  @functools.partial(
      pl.kernel,
      mesh=mesh,
      out_type=jax.ShapeDtypeStruct((_B * 2,), jnp.float32),
      scratch_types=[
          pltpu.VMEM((_STAGE_IROWS, _IMINOR), jnp.int32),
          pltpu.VMEM((_IROWS, _IMINOR, _D), jnp.float32),
          pltpu.VMEM((80,), jnp.float32),
          pltpu.VMEM((2 * _BPW,), jnp.float32),
          pltpu.SemaphoreType.DMA,
      ],
      compiler_params=pltpu.CompilerParams(
          needs_layout_passes=False, use_tc_tiling_on_sc=False),
  )
  def k(tok_hbm, emb_hbm, wb_hbm, out_hbm, idx_v, rows_v, wb_v, out_v, sem):
    cid = lax.axis_index("c")
    sid = lax.axis_index("s")
    wid = sid * _NC + cid

    pltpu.sync_copy(wb_hbm, wb_v)
    w00 = wb_v[pl.ds(0, 16)]
    w01 = wb_v[pl.ds(16, 16)]
    w10 = wb_v[pl.ds(32, 16)]
    w11 = wb_v[pl.ds(48, 16)]
    bvec = wb_v[pl.ds(64, 16)]
    bias0 = bvec[0]
    bias1 = bvec[1]
    lane = lax.iota(jnp.int32, 16)
    inv_h = jnp.float32(1.0 / _H)

    def chunk_body(g, carry):
      gg = g // _CHUNKS_PER_STAGE
      g2 = g % _CHUNKS_PER_STAGE

      @pl.when(g2 == 0)
      def _stage():
        row0 = wid * _IDX_PER_W + gg * _STAGE_IROWS
        pltpu.sync_copy(tok_hbm.at[pl.ds(row0, _STAGE_IROWS)], idx_v)

      def fire(j, c):
        pltpu.async_copy(emb_hbm.at[idx_v.at[g2 * _IROWS + j]], rows_v.at[j],
                         sem)
        return c

      lax.fori_loop(0, _IROWS, fire, 0)

      def drain(j, c):
        pltpu.make_async_copy(emb_hbm.at[idx_v.at[g2 * _IROWS + j]],
                              rows_v.at[j], sem).wait()
        return c

      lax.fori_loop(0, _IROWS, drain, 0)

      ov0 = jnp.zeros((16,), jnp.float32)
      ov1 = jnp.zeros((16,), jnp.float32)
      for b in range(_CROWS):
        t0 = b * _H
        j0, k0 = divmod(t0, _IMINOR)

        def tok_body(l, c):
          a0, a1, j, kk = c
          a0 = a0 + rows_v[j, kk, pl.ds(0, 16)]
          a1 = a1 + rows_v[j, kk, pl.ds(16, 16)]
          kk = kk + 1
          wrap = kk == _IMINOR
          j = jnp.where(wrap, j + 1, j)
          kk = jnp.where(wrap, 0, kk)
          return a0, a1, j, kk

        z = jnp.zeros((16,), jnp.float32)
        a0, a1, _, _ = lax.fori_loop(
            0, _H, tok_body, (z, z, jnp.int32(j0), jnp.int32(k0)))
        m0 = a0 * inv_h
        m1 = a1 * inv_h
        o0 = jnp.sum(m0 * w00) + jnp.sum(m1 * w01) + bias0
        o1 = jnp.sum(m0 * w10) + jnp.sum(m1 * w11) + bias1
        pos = (b % 8) * 2
        if b < 8:
          ov0 = jnp.where(lane == pos, o0, ov0)
          ov0 = jnp.where(lane == pos + 1, o1, ov0)
        else:
          ov1 = jnp.where(lane == pos, o0, ov1)
          ov1 = jnp.where(lane == pos + 1, o1, ov1)
      out_v[pl.ds(g * 2 * _CROWS, 16)] = ov0
      out_v[pl.ds(g * 2 * _CROWS + 16, 16)] = ov1
      return carry

    lax.fori_loop(0, _NCHUNK, chunk_body, 0)
    pltpu.sync_copy(out_v, out_hbm.at[pl.ds(wid * 2 * _BPW, 2 * _BPW)])

  return k


_sc_kernel = _make_sc_kernel()


@jax.jit
def kernel(token_index, emb_table, fc_w, fc_b):
  tok = token_index.astype(jnp.int32).reshape(_IDX_ROWS_TOTAL, _IMINOR)
  wb = jnp.concatenate(
      [fc_w.reshape(-1), fc_b, jnp.zeros((80 - _D * 2 - 2,), jnp.float32)])
  out_flat = _sc_kernel(tok, emb_table, wb)
  return out_flat.reshape(_B, 2)
